# baseline (speedup 1.0000x reference)
import functools

import jax
import jax.numpy as jnp
from jax import lax
from jax.experimental import pallas as pl
from jax.experimental.pallas import tpu as pltpu
from jax.experimental.pallas import tpu_sc as plsc

DIM = 64
EPS = 1e-05
NC, NS = 2, 16
NW = NC * NS
CHUNK = 128
L = 16
NV = DIM // L
NG = CHUNK // L


def _rsqrt(a):
    i = lax.bitcast_convert_type(a, jnp.int32)
    i = jnp.full((L,), 0x5F3759DF, jnp.int32) - lax.shift_right_arithmetic(i, 1)
    y = lax.bitcast_convert_type(i, jnp.float32)
    xh = a * 0.5
    y = y * (1.5 - xh * y * y)
    y = y * (1.5 - xh * y * y)
    y = y * (1.5 - xh * y * y)
    return y


def _make_call(nchunk):
    rows_per_w = nchunk * CHUNK
    total = NW * rows_per_w
    mesh = plsc.VectorSubcoreMesh(core_axis_name="c", subcore_axis_name="s")

    @functools.partial(
        pl.kernel,
        mesh=mesh,
        compiler_params=pltpu.CompilerParams(
            needs_layout_passes=False, use_tc_tiling_on_sc=False),
        out_type=jax.ShapeDtypeStruct((total, DIM), jnp.float32),
        scratch_types=[
            pltpu.VMEM((nchunk, CHUNK), jnp.int32),
            pltpu.VMEM((CHUNK, DIM), jnp.float32),
            pltpu.VMEM((CHUNK, DIM), jnp.float32),
            pltpu.VMEM((DIM, CHUNK), jnp.float32),
            pltpu.VMEM((CHUNK, DIM), jnp.float32),
            pltpu.VMEM((CHUNK, DIM), jnp.float32),
            pltpu.VMEM((2, DIM), jnp.float32),
            pltpu.VMEM((DIM, L), jnp.float32),
            pltpu.VMEM((DIM, L), jnp.float32),
            pltpu.SemaphoreType.DMA,
            pltpu.SemaphoreType.DMA,
            pltpu.SemaphoreType.DMA,
            pltpu.SemaphoreType.DMA,
        ],
    )
    def call(idx_hbm, table_hbm, scale_hbm, bias_hbm, out_hbm,
             idx_v, xb0, xb1, xt_v, ob0, ob1, sb_v, sbc_v, bbc_v,
             sg0, sg1, so0, so1):
        cid = lax.axis_index("c")
        sid = lax.axis_index("s")
        wid = sid * NC + cid

        pltpu.sync_copy(idx_hbm.at[wid], idx_v)
        pltpu.sync_copy(scale_hbm, sb_v.at[0])
        pltpu.sync_copy(bias_hbm, sb_v.at[1])

        lanes = lax.iota(jnp.int32, L)
        ri = [lanes + (L * k) for k in range(NV)]
        rg = [lanes + (L * g) for g in range(NG)]
        ones = jnp.full((L,), 1, jnp.int32)
        zeros = jnp.full((L,), 0, jnp.int32)

        for k in range(NV):
            svk = sb_v[0, pl.ds(L * k, L)]
            bvk = sb_v[1, pl.ds(L * k, L)]
            col = zeros
            for _ in range(L):
                plsc.store_scatter(sbc_v, [ri[k], col], svk)
                plsc.store_scatter(bbc_v, [ri[k], col], bvk)
                col = col + ones

        out_base = wid * rows_per_w

        def compute(xb, ob):
            def t_body(i, col):
                for j in range(4):
                    r = i * 4 + j
                    for k in range(NV):
                        plsc.store_scatter(
                            xt_v, [ri[k], col], xb[r, pl.ds(L * k, L)])
                    col = col + ones
                return col
            lax.fori_loop(0, CHUNK // 4, t_body, zeros)

            def s_body(i, accs):
                s, q = accs
                for j in range(4):
                    d = i * 4 + j
                    s = list(s)
                    q = list(q)
                    for g in range(NG):
                        v = xt_v[d, pl.ds(L * g, L)]
                        s[g] = s[g] + v
                        q[g] = q[g] + v * v
                    s = tuple(s)
                    q = tuple(q)
                return (s, q)
            z = tuple(jnp.zeros((L,), jnp.float32) for _ in range(NG))
            s, q = lax.fori_loop(0, DIM // 4, s_body, (z, z))

            rs = []
            mr = []
            for g in range(NG):
                mean = s[g] * (1.0 / DIM)
                var = q[g] * (1.0 / DIM) - mean * mean
                r_ = _rsqrt(var + EPS)
                rs.append(r_)
                mr.append(mean * r_)

            def n_body(i, col):
                for j in range(4):
                    d = i * 4 + j
                    sv = sbc_v[d, pl.ds(0, L)]
                    bv = bbc_v[d, pl.ds(0, L)]
                    for g in range(NG):
                        x = xt_v[d, pl.ds(L * g, L)]
                        o = x * (rs[g] * sv) + (bv - mr[g] * sv)
                        plsc.store_scatter(ob, [rg[g], col], o)
                    col = col + ones
                return col
            lax.fori_loop(0, DIM // 4, n_body, zeros)

        def pair_body(p, carry):
            c0 = 2 * p
            c1 = c0 + 1
            pltpu.async_copy(table_hbm.at[idx_v.at[c1]], xb1, sg1)
            pltpu.make_async_copy(table_hbm.at[idx_v.at[c0]], xb0, sg0).wait()

            @pl.when(p > 0)
            def _():
                pltpu.make_async_copy(
                    ob0, out_hbm.at[pl.ds(out_base + (c0 - 2) * CHUNK, CHUNK)],
                    so0).wait()
            compute(xb0, ob0)
            pltpu.async_copy(
                ob0, out_hbm.at[pl.ds(out_base + c0 * CHUNK, CHUNK)], so0)

            @pl.when(c1 + 1 < nchunk)
            def _():
                pltpu.async_copy(table_hbm.at[idx_v.at[c1 + 1]], xb0, sg0)

            pltpu.make_async_copy(table_hbm.at[idx_v.at[c1]], xb1, sg1).wait()

            @pl.when(p > 0)
            def _():
                pltpu.make_async_copy(
                    ob1, out_hbm.at[pl.ds(out_base + (c1 - 2) * CHUNK, CHUNK)],
                    so1).wait()
            compute(xb1, ob1)
            pltpu.async_copy(
                ob1, out_hbm.at[pl.ds(out_base + c1 * CHUNK, CHUNK)], so1)
            return carry

        pltpu.async_copy(table_hbm.at[idx_v.at[0]], xb0, sg0)
        lax.fori_loop(0, nchunk // 2, pair_body, 0)
        pltpu.make_async_copy(
            ob0, out_hbm.at[pl.ds(out_base + (nchunk - 2) * CHUNK, CHUNK)],
            so0).wait()
        pltpu.make_async_copy(
            ob1, out_hbm.at[pl.ds(out_base + (nchunk - 1) * CHUNK, CHUNK)],
            so1).wait()

    return call


_CALLS = {}


def kernel(inputs, emb_weight, ln_scale, ln_bias):
    b, s = inputs.shape
    total = b * s
    assert total % (NW * 2 * CHUNK) == 0
    nchunk = total // (NW * CHUNK)
    if nchunk not in _CALLS:
        _CALLS[nchunk] = _make_call(nchunk)
    idx = inputs.astype(jnp.int32).reshape(NW, nchunk, CHUNK)
    out = _CALLS[nchunk](idx, emb_weight, ln_scale, ln_bias)
    return out.reshape(b, s, DIM)


# speedup vs baseline: 1.6397x; 1.6397x over previous
import functools

import jax
import jax.numpy as jnp
from jax import lax
from jax.experimental import pallas as pl
from jax.experimental.pallas import tpu as pltpu
from jax.experimental.pallas import tpu_sc as plsc

DIM = 64
EPS = 1e-05
NC, NS = 2, 16
NW = NC * NS
CHUNK = 128
L = 16
NV = DIM // L
UNROLL = 4


def _rsqrt(a):
    i = lax.bitcast_convert_type(a, jnp.int32)
    i = jnp.int32(0x5F3759DF) - lax.shift_right_arithmetic(i, 1)
    y = lax.bitcast_convert_type(i, jnp.float32)
    xh = a * 0.5
    y = y * (1.5 - xh * y * y)
    y = y * (1.5 - xh * y * y)
    y = y * (1.5 - xh * y * y)
    return y


def _make_call(nchunk):
    rows_per_w = nchunk * CHUNK
    total = NW * rows_per_w
    mesh = plsc.VectorSubcoreMesh(core_axis_name="c", subcore_axis_name="s")

    @functools.partial(
        pl.kernel,
        mesh=mesh,
        compiler_params=pltpu.CompilerParams(
            needs_layout_passes=False, use_tc_tiling_on_sc=False),
        out_type=jax.ShapeDtypeStruct((total, DIM), jnp.float32),
        scratch_types=[
            pltpu.VMEM((nchunk, CHUNK), jnp.int32),
            pltpu.VMEM((CHUNK, DIM), jnp.float32),
            pltpu.VMEM((CHUNK, DIM), jnp.float32),
            pltpu.VMEM((CHUNK, DIM), jnp.float32),
            pltpu.VMEM((CHUNK, DIM), jnp.float32),
            pltpu.VMEM((2, DIM), jnp.float32),
            pltpu.SemaphoreType.DMA,
            pltpu.SemaphoreType.DMA,
            pltpu.SemaphoreType.DMA,
            pltpu.SemaphoreType.DMA,
        ],
    )
    def call(idx_hbm, table_hbm, scale_hbm, bias_hbm, out_hbm,
             idx_v, xb0, xb1, ob0, ob1, sb_v, sg0, sg1, so0, so1):
        cid = lax.axis_index("c")
        sid = lax.axis_index("s")
        wid = sid * NC + cid

        pltpu.sync_copy(idx_hbm.at[wid], idx_v)
        pltpu.sync_copy(scale_hbm, sb_v.at[0])
        pltpu.sync_copy(bias_hbm, sb_v.at[1])
        sv = [sb_v[0, pl.ds(L * k, L)] for k in range(NV)]
        bv = [sb_v[1, pl.ds(L * k, L)] for k in range(NV)]
        out_base = wid * rows_per_w

        def ln_rows(xb, ob, i):
            xs = []
            ss = []
            qs = []
            for j in range(UNROLL):
                r = i * UNROLL + j
                x = [xb[r, pl.ds(L * k, L)] for k in range(NV)]
                s = (x[0] + x[1]) + (x[2] + x[3])
                q = (x[0] * x[0] + x[1] * x[1]) + (x[2] * x[2] + x[3] * x[3])
                xs.append(x)
                ss.append(jnp.sum(s))
                qs.append(jnp.sum(q))
            coefs = []
            for j in range(UNROLL):
                mean = ss[j] * (1.0 / DIM)
                var = qs[j] * (1.0 / DIM) - mean * mean
                rinv = _rsqrt(var + EPS)
                mr = mean * rinv
                coefs.append((lax.broadcast_in_dim(rinv, (L,), ()),
                              lax.broadcast_in_dim(mr, (L,), ())))
            for j in range(UNROLL):
                r = i * UNROLL + j
                rsj, mrj = coefs[j]
                for k in range(NV):
                    o = xs[j][k] * (sv[k] * rsj) + (bv[k] - sv[k] * mrj)
                    ob[r, pl.ds(L * k, L)] = o

        def compute(xb, ob):
            lax.fori_loop(0, CHUNK // UNROLL,
                          lambda i, c: (ln_rows(xb, ob, i), c)[1], 0)

        def pair_body(p, carry):
            c0 = 2 * p
            c1 = c0 + 1
            pltpu.async_copy(table_hbm.at[idx_v.at[c1]], xb1, sg1)
            pltpu.make_async_copy(table_hbm.at[idx_v.at[c0]], xb0, sg0).wait()

            @pl.when(p > 0)
            def _():
                pltpu.make_async_copy(
                    ob0, out_hbm.at[pl.ds(out_base + (c0 - 2) * CHUNK, CHUNK)],
                    so0).wait()
            compute(xb0, ob0)
            pltpu.async_copy(
                ob0, out_hbm.at[pl.ds(out_base + c0 * CHUNK, CHUNK)], so0)

            @pl.when(c1 + 1 < nchunk)
            def _():
                pltpu.async_copy(table_hbm.at[idx_v.at[c1 + 1]], xb0, sg0)

            pltpu.make_async_copy(table_hbm.at[idx_v.at[c1]], xb1, sg1).wait()

            @pl.when(p > 0)
            def _():
                pltpu.make_async_copy(
                    ob1, out_hbm.at[pl.ds(out_base + (c1 - 2) * CHUNK, CHUNK)],
                    so1).wait()
            compute(xb1, ob1)
            pltpu.async_copy(
                ob1, out_hbm.at[pl.ds(out_base + c1 * CHUNK, CHUNK)], so1)
            return carry

        pltpu.async_copy(table_hbm.at[idx_v.at[0]], xb0, sg0)
        lax.fori_loop(0, nchunk // 2, pair_body, 0)
        pltpu.make_async_copy(
            ob0, out_hbm.at[pl.ds(out_base + (nchunk - 2) * CHUNK, CHUNK)],
            so0).wait()
        pltpu.make_async_copy(
            ob1, out_hbm.at[pl.ds(out_base + (nchunk - 1) * CHUNK, CHUNK)],
            so1).wait()

    return call


_CALLS = {}


def kernel(inputs, emb_weight, ln_scale, ln_bias):
    b, s = inputs.shape
    total = b * s
    assert total % (NW * 2 * CHUNK) == 0
    nchunk = total // (NW * CHUNK)
    if nchunk not in _CALLS:
        _CALLS[nchunk] = _make_call(nchunk)
    idx = inputs.astype(jnp.int32).reshape(NW, nchunk, CHUNK)
    out = _CALLS[nchunk](idx, emb_weight, ln_scale, ln_bias)
    return out.reshape(b, s, DIM)
